# Initial kernel scaffold; baseline (speedup 1.0000x reference)
import jax
import jax.numpy as jnp
from jax.experimental import pallas as pl


def kernel(x, edge_index, W1, b1, W2, b2, W3, b3, W4, b4, W5, b5):
    raise NotImplementedError("write your pallas kernel here")


# speedup vs baseline: 3.1164x; 3.1164x over previous
import functools

import jax
import jax.numpy as jnp
from jax import lax
from jax.experimental import pallas as pl
from jax.experimental.pallas import tpu as pltpu
from jax.experimental.pallas import tpu_sc as plsc

F32 = jnp.float32

_NC = 2
_NS = 16
_BATCH = 128
_WROWS = 4096
_WSTEP = 3968
_NWIN = 3
_JUNK = (_WSTEP + 4) * 8
_GZERO = (_WSTEP + 64) * 8


@functools.lru_cache(maxsize=None)
def _make_agg(NP, R):
    win_rows = [min(_WSTEP, NP - w * _WSTEP) for w in range(_NWIN)]

    mesh = plsc.VectorSubcoreMesh(core_axis_name="c", subcore_axis_name="s")

    @functools.partial(
        pl.kernel,
        out_type=jax.ShapeDtypeStruct((_NC, _NWIN, _WROWS, 16), F32),
        mesh=mesh,
        scratch_types=[
            pltpu.VMEM((_BATCH,), jnp.int32),
            pltpu.VMEM((_BATCH,), jnp.int32),
            pltpu.VMEM((_BATCH,), jnp.int32),
            pltpu.VMEM((_BATCH,), jnp.int32),
            pltpu.VMEM((_BATCH,), jnp.int32),
            pltpu.VMEM((_BATCH,), jnp.int32),
            pltpu.VMEM((_BATCH, 16), F32),
            pltpu.VMEM((_BATCH, 16), F32),
            pltpu.VMEM((_BATCH, 16), F32),
            pltpu.VMEM_SHARED((_WROWS, 16), F32),
            pltpu.VMEM_SHARED((_WROWS, 16), F32),
            pltpu.VMEM_SHARED((_WROWS, 16), F32),
            pltpu.VMEM_SHARED((_WROWS, 16), F32),
            pltpu.VMEM_SHARED((_WROWS, 16), F32),
            pltpu.VMEM_SHARED((_WROWS, 16), F32),
            pltpu.SemaphoreType.DMA,
        ],
    )
    def agg(tab_hbm, zero_hbm, s0_hbm, s1_hbm, s2_hbm, d0_hbm, d1_hbm, d2_hbm,
            out_hbm, s0_i, s1_i, s2_i, d0_i, d1_i, d2_i, m0, m1, m2,
            t0, t1, t2, a0, a1, a2, sem):
        c = lax.axis_index("c")
        s = lax.axis_index("s")
        wid = s * _NC + c

        for w, tw in enumerate((t0, t1, t2)):
            rpt = win_rows[w] // _NS
            pltpu.sync_copy(tab_hbm.at[pl.ds(w * _WSTEP + s * rpt, rpt)],
                            tw.at[pl.ds(s * rpt, rpt)])
            zpt = (_WROWS - win_rows[w]) // _NS
            pltpu.sync_copy(zero_hbm.at[pl.ds(s * zpt, zpt)],
                            tw.at[pl.ds(win_rows[w] + s * zpt, zpt)])
        rptw = _WROWS // _NS
        r0 = s * rptw
        for aw in (a0, a1, a2):
            pltpu.sync_copy(zero_hbm.at[pl.ds(r0, rptw)], aw.at[pl.ds(r0, rptw)])
        plsc.subcore_barrier()

        def body(j, carry):
            pltpu.sync_copy(s0_hbm.at[wid, j], s0_i)
            pltpu.sync_copy(s1_hbm.at[wid, j], s1_i)
            pltpu.sync_copy(s2_hbm.at[wid, j], s2_i)
            pltpu.sync_copy(d0_hbm.at[wid, j], d0_i)
            pltpu.sync_copy(d1_hbm.at[wid, j], d1_i)
            pltpu.sync_copy(d2_hbm.at[wid, j], d2_i)
            pltpu.async_copy(t0.at[s0_i], m0, sem).wait()
            pltpu.async_copy(t1.at[s1_i], m1, sem).wait()
            pltpu.async_copy(t2.at[s2_i], m2, sem).wait()
            for mw in (m0, m1, m2):
                pltpu.sync_copy(mw, a0.at[d0_i], add=True)
                pltpu.sync_copy(mw, a1.at[d1_i], add=True)
                pltpu.sync_copy(mw, a2.at[d2_i], add=True)
            plsc.subcore_barrier()
            return carry
        lax.fori_loop(0, R, body, 0)

        plsc.subcore_barrier()
        for w, aw in enumerate((a0, a1, a2)):
            pltpu.sync_copy(aw.at[pl.ds(r0, rptw)],
                            out_hbm.at[c, w, pl.ds(r0, rptw)])

    return agg


def _tc(body, out_shape, *args):
    return pl.pallas_call(body, out_shape=out_shape)(*args)


def _assemble(parts, NP):
    s = parts[0] + parts[1]
    pieces = []
    left = NP
    for w in range(_NWIN):
        take = min(_WSTEP, left)
        if take > 0:
            pieces.append(s[w, :take])
        left -= take
    return jnp.concatenate(pieces, axis=0)


def _linear_body(x_ref, w_ref, o_ref):
    o_ref[...] = jnp.dot(x_ref[...], w_ref[...], preferred_element_type=F32)


def _make_g_body(NP):
    def body(dp_ref, p_ref, dinv_ref, g_ref):
        deg = _assemble(dp_ref[...], NP) + 1.0
        dinv = lax.rsqrt(deg)
        dinv_ref[...] = dinv
        g_ref[...] = dinv * p_ref[...]
    return body


def _make_layer1_body(NP):
    def body(parts_ref, g_ref, dinv_ref, b_ref, o_ref):
        dinv = dinv_ref[...]
        agg = dinv * (_assemble(parts_ref[...], NP) + g_ref[...])
        h = jnp.maximum(agg + b_ref[...], 0.0)
        o_ref[...] = dinv * h
    return body


def _make_mid_body(NP):
    def body(parts_ref, g_ref, dinv_ref, w_ref, b_ref, o_ref):
        dinv = dinv_ref[...]
        agg = dinv * (_assemble(parts_ref[...], NP) + g_ref[...])
        h = jnp.dot(agg, w_ref[...], preferred_element_type=F32) + b_ref[...]
        o_ref[...] = dinv * jnp.maximum(h, 0.0)
    return body


def _make_final_body(NP, n_valid):
    def body(parts_ref, g_ref, dinv_ref, w_ref, b_ref, o_ref):
        agg = dinv_ref[...] * (_assemble(parts_ref[...], NP) + g_ref[...])
        logits = jnp.dot(agg, w_ref[...], preferred_element_type=F32) + b_ref[...]
        m = jnp.max(logits, axis=1, keepdims=True)
        e = jnp.exp(logits - m)
        sm = e / jnp.sum(e, axis=1, keepdims=True)
        o_ref[...] = sm[:n_valid]
    return body


def _win_offsets(idx, junk_offsets):
    outs = []
    for w in range(_NWIN):
        rel = idx - w * _WSTEP
        ok = (rel >= 0) & (rel < _WSTEP)
        outs.append(jnp.where(ok, rel * 8, junk_offsets).astype(jnp.int32))
    return outs


def kernel(x, edge_index, W1, b1, W2, b2, W3, b3, W4, b4, W5, b5):
    n, d = x.shape
    e = edge_index.shape[1]
    NW = _NC * _NS

    per_tile = -(-e // NW)
    per_tile = -(-per_tile // _BATCH) * _BATCH
    R = per_tile // _BATCH
    epad = per_tile * NW
    NP = -(-n // (_NS * 8)) * (_NS * 8)
    assert NP <= _NWIN * _WSTEP

    srce = jnp.concatenate(
        [edge_index[0], jnp.full((epad - e,), n, jnp.int32)])
    dste = jnp.concatenate(
        [edge_index[1], jnp.full((epad - e,), _NWIN * _WSTEP, jnp.int32)])
    wid_e = jnp.arange(epad, dtype=jnp.int32) // per_tile
    gzero_e = _GZERO + wid_e * 8
    junk_e = _JUNK + wid_e * 8
    swin = [o.reshape(NW, R, _BATCH) for o in _win_offsets(srce, gzero_e)]
    dwin = [o.reshape(NW, R, _BATCH) for o in _win_offsets(dste, junk_e)]

    agg0 = _make_agg(NP, R)
    zero_t = jnp.zeros((_WROWS, 16), F32)
    agg = lambda tab: agg0(tab, zero_t, swin[0], swin[1], swin[2],
                           dwin[0], dwin[1], dwin[2])

    ones_t = jnp.zeros((NP, 16), F32).at[:n].set(1.0)
    deg_parts = agg(ones_t)

    xp = jnp.zeros((NP, d), F32).at[:n].set(x)
    p1 = _tc(_linear_body, jax.ShapeDtypeStruct((NP, 16), F32), xp, W1)

    dinv, g = _tc(
        _make_g_body(NP),
        (jax.ShapeDtypeStruct((NP, 16), F32), jax.ShapeDtypeStruct((NP, 16), F32)),
        deg_parts, p1)

    parts = agg(g)
    g = _tc(_make_layer1_body(NP), jax.ShapeDtypeStruct((NP, 16), F32),
            parts, g, dinv, b1.reshape(1, 16))

    for (W, b) in ((W2, b2), (W3, b3), (W4, b4)):
        parts = agg(g)
        g = _tc(_make_mid_body(NP), jax.ShapeDtypeStruct((NP, 16), F32),
                parts, g, dinv, W, b.reshape(1, 16))

    parts = agg(g)
    out = _tc(_make_final_body(NP, n), jax.ShapeDtypeStruct((n, 32), F32),
              parts, g, dinv, W5, b5.reshape(1, 32))
    return out
